# baseline (speedup 1.0000x reference)
import jax
import jax.numpy as jnp
from jax import lax
from jax.experimental import pallas as pl
from jax.experimental.pallas import tpu as pltpu
from jax.experimental.pallas import tpu_sc as plsc

N = 1_000_000
C = 16
TS = 1024
R = 103
NSLAB = 10
ACC_ROWS = (R + 2) * TS
S = 1024
T_CHUNKS = 62
E = 128
Q = 1280
TRASH = Q - 16
ZROWS = 480
ACC_PER_TILE = ACC_ROWS // 16
PER_TILE = 62496


def _step01(x):
    return jnp.minimum(jnp.maximum(x + 1, 0), 1)


def _splat_body(values_hbm, uv_hbm, out_hbm,
                uv_buf, pid_q, fx_q, fy_q, idx_q,
                rows_v, out_data, out_idx, sem_scan, sem_g, sem_s):
    cid = lax.axis_index("c")
    sid = lax.axis_index("s")

    start = sid * PER_TILE
    count = jnp.where(sid == 15, N - 15 * PER_TILE, PER_TILE)
    end = start + count
    iota16 = lax.iota(jnp.int32, 16)
    zf16 = jnp.zeros((16,), jnp.float32)
    zi16 = jnp.zeros((16,), jnp.int32)

    def chunk_start(i):
        return jnp.minimum(start + i * S, end - S)

    def issue_scan(i, par):
        cs = chunk_start(i)
        pltpu.async_copy(uv_hbm.at[pl.ds(2 * cs, 2 * S)],
                         uv_buf.at[pl.ds(par * 2 * S, 2 * S)], sem_scan)

    def wait_scan(i, par):
        cs = chunk_start(i)
        pltpu.make_async_copy(uv_hbm.at[pl.ds(2 * cs, 2 * S)],
                              uv_buf.at[pl.ds(par * 2 * S, 2 * S)],
                              sem_scan).wait()

    def run_pass(acc, lo, hi):
        def _zb(i, _):
            out_data[i, :] = zf16
            return 0
        lax.fori_loop(0, ZROWS, _zb, 0)
        zdescs = []
        for k in range(ACC_PER_TILE // ZROWS):
            zdescs.append(pltpu.async_copy(
                out_data.at[pl.ds(0, ZROWS)],
                acc.at[pl.ds(sid * ACC_PER_TILE + k * ZROWS, ZROWS)],
                sem_scan))
        for d in zdescs:
            d.wait()
        plsc.subcore_barrier()

        def do_emit(e, pend):
            head = e * E
            gd = pltpu.async_copy(values_hbm.at[pid_q.at[pl.ds(head, E)]],
                                  rows_v, sem_g)

            @pl.when(pend == 1)
            def _drain():
                for q in range(4):
                    pltpu.make_async_copy(out_data.at[pl.ds(q * E, E)],
                                          acc.at[out_idx.at[q]], sem_s).wait()
            gd.wait()

            def per_group(g, _):
                base = head + g * 16
                idxv = idx_q[pl.ds(base, 16)]
                out_idx[0, pl.ds(g * 16, 16)] = idxv
                out_idx[1, pl.ds(g * 16, 16)] = idxv + 1
                out_idx[2, pl.ds(g * 16, 16)] = idxv + TS
                out_idx[3, pl.ds(g * 16, 16)] = idxv + (TS + 1)
                fxg = fx_q[pl.ds(base, 16)]
                fyg = fy_q[pl.ds(base, 16)]
                for j in range(16):
                    r = g * 16 + j
                    fx_s = fxg[j]
                    fy_s = fyg[j]
                    vrow = rows_v[r, :]
                    b = vrow * fy_s
                    a = vrow - b
                    t10 = a * fx_s
                    t00 = a - t10
                    t11 = b * fx_s
                    t01 = b - t11
                    out_data[r, :] = t00
                    out_data[E + r, :] = t10
                    out_data[2 * E + r, :] = t01
                    out_data[3 * E + r, :] = t11
                return 0
            lax.fori_loop(0, E // 16, per_group, 0)

            for q in range(4):
                pltpu.async_copy(out_data.at[pl.ds(q * E, E)],
                                 acc.at[out_idx.at[q]], sem_s, add=True)
            return jnp.int32(1)

        def scan_chunk(i, carry):
            tail, pend = carry
            par = lax.rem(i, 2)
            wait_scan(i, par)
            issue_scan(jnp.minimum(i + 1, T_CHUNKS - 1), lax.rem(i + 1, 2))
            cstart = chunk_start(i)
            logical = start + i * S
            boff2 = par * 2 * S

            def group(g4, tail):
                datas = []
                for uu in range(4):
                    g = g4 * 4 + uu
                    pbase = boff2 + g * 32
                    u16 = plsc.load_gather(uv_buf, [pbase + 2 * iota16])
                    v16 = plsc.load_gather(uv_buf, [pbase + 2 * iota16 + 1])
                    ux = u16 * jnp.float32(TS - 1)
                    vy = v16 * jnp.float32(TS - 1)
                    x0 = jnp.minimum(ux.astype(jnp.int32), TS - 2)
                    y0 = jnp.minimum(vy.astype(jnp.int32), TS - 2)
                    fxv = ux - x0.astype(jnp.float32)
                    fyv = vy - y0.astype(jnp.float32)
                    pidv = cstart + g * 16 + iota16
                    m01 = (_step01(y0 - (lo - 1)) * _step01((hi - 1) - y0)
                           * _step01(pidv - logical))
                    idxv = (y0 - (lo - 1)) * TS + x0
                    incl = plsc.cumsum(m01)
                    datas.append((m01, incl, pidv, fxv, fyv, idxv))
                for m01, incl, pidv, fxv, fyv, idxv in datas:
                    pos_sel = tail - 1 + incl
                    pos = TRASH + m01 * (pos_sel - TRASH)
                    plsc.store_scatter(pid_q, [pos], pidv)
                    plsc.store_scatter(fx_q, [pos], fxv)
                    plsc.store_scatter(fy_q, [pos], fyv)
                    plsc.store_scatter(idx_q, [pos], idxv)
                    tail = tail + incl[15]
                return tail
            tail = lax.fori_loop(0, S // 64, group, tail)

            nblk = tail // E
            pend = lax.fori_loop(0, nblk, do_emit, pend)
            for k in range(E // 16):
                off = nblk * E + k * 16
                pv = pid_q[pl.ds(off, 16)]
                fxv = fx_q[pl.ds(off, 16)]
                fyv = fy_q[pl.ds(off, 16)]
                iv = idx_q[pl.ds(off, 16)]
                pid_q[pl.ds(k * 16, 16)] = pv
                fx_q[pl.ds(k * 16, 16)] = fxv
                fy_q[pl.ds(k * 16, 16)] = fyv
                idx_q[pl.ds(k * 16, 16)] = iv
            return (tail - nblk * E, pend)

        issue_scan(0, 0)
        tail, pend = lax.fori_loop(0, T_CHUNKS, scan_chunk,
                                   (jnp.int32(0), jnp.int32(0)))
        wait_scan(T_CHUNKS - 1, lax.rem(jnp.int32(T_CHUNKS), 2))

        for k in range(E // 16):
            pid_q[pl.ds(tail + k * 16, 16)] = zi16
            fx_q[pl.ds(tail + k * 16, 16)] = zf16
            fy_q[pl.ds(tail + k * 16, 16)] = zf16
            idx_q[pl.ds(tail + k * 16, 16)] = zi16
        pend = lax.fori_loop(0, (tail + E - 1) // E, do_emit, pend)

        @pl.when(pend == 1)
        def _final_drain():
            for q in range(4):
                pltpu.make_async_copy(out_data.at[pl.ds(q * E, E)],
                                      acc.at[out_idx.at[q]], sem_s).wait()

        plsc.subcore_barrier()

    def run_slab(acc, rows, lo, hi):
        run_pass(acc, lo, hi)
        wrows = rows * (TS // 16)
        src = TS + sid * wrows
        dst = lo * TS + sid * wrows
        pltpu.sync_copy(acc.at[pl.ds(src, wrows)],
                        out_hbm.at[pl.ds(dst, wrows)])
        plsc.subcore_barrier()

    def body(acc):
        for p in range(5):
            slab = 2 * p + cid
            lo = slab * R
            hi = jnp.minimum(lo + R, TS)

            @pl.when(slab < NSLAB - 1)
            def _full():
                run_slab(acc, R, lo, hi)

            @pl.when(slab == NSLAB - 1)
            def _last():
                run_slab(acc, TS - (NSLAB - 1) * R, lo, hi)

    return body


def _splat_kernel(values_hbm, uv_hbm, out_hbm,
                  uv_buf, pid_q, fx_q, fy_q, idx_q,
                  rows_v, out_data, out_idx, acc, sem_scan, sem_g, sem_s):
    body = _splat_body(values_hbm, uv_hbm, out_hbm,
                       uv_buf, pid_q, fx_q, fy_q, idx_q,
                       rows_v, out_data, out_idx, sem_scan, sem_g, sem_s)
    body(acc)


def kernel(values_tensor, uv_tensor, texture_size):
    del texture_size
    uv_flat = uv_tensor.reshape(2 * N)

    mesh = plsc.VectorSubcoreMesh(core_axis_name="c", subcore_axis_name="s")
    fn = pl.kernel(
        _splat_kernel,
        out_type=jax.ShapeDtypeStruct((TS * TS, C), jnp.float32),
        mesh=mesh,
        scratch_types=[
            pltpu.VMEM((4 * S,), jnp.float32),
            pltpu.VMEM((Q,), jnp.int32),
            pltpu.VMEM((Q,), jnp.float32),
            pltpu.VMEM((Q,), jnp.float32),
            pltpu.VMEM((Q,), jnp.int32),
            pltpu.VMEM((E, C), jnp.float32),
            pltpu.VMEM((4 * E, C), jnp.float32),
            pltpu.VMEM((4, E), jnp.int32),
            pltpu.VMEM_SHARED((ACC_ROWS, C), jnp.float32),
            pltpu.SemaphoreType.DMA,
            pltpu.SemaphoreType.DMA,
            pltpu.SemaphoreType.DMA,
        ],
        compiler_params=pltpu.CompilerParams(use_tc_tiling_on_sc=False,
                                             needs_layout_passes=False),
    )
    tex = fn(values_tensor, uv_flat)
    return tex.reshape(TS, TS, C)


# speedup vs baseline: 1.7219x; 1.7219x over previous
import jax
import jax.numpy as jnp
from jax import lax
from jax.experimental import pallas as pl
from jax.experimental.pallas import tpu as pltpu
from jax.experimental.pallas import tpu_sc as plsc

N = 1_000_000
C = 16
TS = 1024
R = 103
NSLAB = 10
ACC_ROWS = (R + 2) * TS
S = 1024
T_CHUNKS = 62
E = 128
Q = 1280
TRASH = Q - 16
ZROWS = 480
ACC_PER_TILE = ACC_ROWS // 16
PER_TILE = 62496


def _step01(x):
    return jnp.minimum(jnp.maximum(x + 1, 0), 1)


def _splat_body(values_hbm, u_hbm, v_hbm, out_hbm,
                u_buf, v_buf, pid_q, fx_q, fy_q, idx_q,
                rows_v, out_data, out_idx, sem_scan, sem_g, sem_s):
    cid = lax.axis_index("c")
    sid = lax.axis_index("s")

    start = sid * PER_TILE
    count = jnp.where(sid == 15, N - 15 * PER_TILE, PER_TILE)
    end = start + count
    iota16 = lax.iota(jnp.int32, 16)
    zf16 = jnp.zeros((16,), jnp.float32)
    zi16 = jnp.zeros((16,), jnp.int32)

    def chunk_start(i):
        return jnp.minimum(start + i * S, end - S)

    def issue_scan(i, par):
        cs = chunk_start(i)
        pltpu.async_copy(u_hbm.at[pl.ds(cs, S)],
                         u_buf.at[pl.ds(par * S, S)], sem_scan)
        pltpu.async_copy(v_hbm.at[pl.ds(cs, S)],
                         v_buf.at[pl.ds(par * S, S)], sem_scan)

    def wait_scan(i, par):
        cs = chunk_start(i)
        pltpu.make_async_copy(u_hbm.at[pl.ds(cs, S)],
                              u_buf.at[pl.ds(par * S, S)], sem_scan).wait()
        pltpu.make_async_copy(v_hbm.at[pl.ds(cs, S)],
                              v_buf.at[pl.ds(par * S, S)], sem_scan).wait()

    def run_pass(acc, lo, hi):
        def _zb(i, _):
            out_data[i, :] = zf16
            return 0
        lax.fori_loop(0, ZROWS, _zb, 0)
        zdescs = []
        for k in range(ACC_PER_TILE // ZROWS):
            zdescs.append(pltpu.async_copy(
                out_data.at[pl.ds(0, ZROWS)],
                acc.at[pl.ds(sid * ACC_PER_TILE + k * ZROWS, ZROWS)],
                sem_scan))
        for d in zdescs:
            d.wait()
        plsc.subcore_barrier()

        def do_emit(e, pend):
            head = e * E
            gd = pltpu.async_copy(values_hbm.at[pid_q.at[pl.ds(head, E)]],
                                  rows_v, sem_g)

            @pl.when(pend == 1)
            def _drain():
                for q in range(4):
                    pltpu.make_async_copy(out_data.at[pl.ds(q * E, E)],
                                          acc.at[out_idx.at[q]], sem_s).wait()
            gd.wait()

            def per_group(g, _):
                base = head + g * 16
                idxv = idx_q[pl.ds(base, 16)]
                out_idx[0, pl.ds(g * 16, 16)] = idxv
                out_idx[1, pl.ds(g * 16, 16)] = idxv + 1
                out_idx[2, pl.ds(g * 16, 16)] = idxv + TS
                out_idx[3, pl.ds(g * 16, 16)] = idxv + (TS + 1)
                fxg = fx_q[pl.ds(base, 16)]
                fyg = fy_q[pl.ds(base, 16)]
                for j in range(16):
                    r = g * 16 + j
                    fx_s = fxg[j]
                    fy_s = fyg[j]
                    vrow = rows_v[r, :]
                    b = vrow * fy_s
                    a = vrow - b
                    t10 = a * fx_s
                    t00 = a - t10
                    t11 = b * fx_s
                    t01 = b - t11
                    out_data[r, :] = t00
                    out_data[E + r, :] = t10
                    out_data[2 * E + r, :] = t01
                    out_data[3 * E + r, :] = t11
                return 0
            lax.fori_loop(0, E // 16, per_group, 0)

            for q in range(4):
                pltpu.async_copy(out_data.at[pl.ds(q * E, E)],
                                 acc.at[out_idx.at[q]], sem_s, add=True)
            return jnp.int32(1)

        def scan_chunk(i, carry):
            tail, pend = carry
            par = lax.rem(i, 2)
            wait_scan(i, par)
            issue_scan(jnp.minimum(i + 1, T_CHUNKS - 1), lax.rem(i + 1, 2))
            cstart = chunk_start(i)
            logical = start + i * S
            brow = par * S

            def group(g4, tail):
                datas = []
                for uu in range(4):
                    g = g4 * 4 + uu
                    u16 = u_buf[pl.ds(brow + g * 16, 16)]
                    v16 = v_buf[pl.ds(brow + g * 16, 16)]
                    ux = u16 * jnp.float32(TS - 1)
                    vy = v16 * jnp.float32(TS - 1)
                    x0 = jnp.minimum(ux.astype(jnp.int32), TS - 2)
                    y0 = jnp.minimum(vy.astype(jnp.int32), TS - 2)
                    fxv = ux - x0.astype(jnp.float32)
                    fyv = vy - y0.astype(jnp.float32)
                    pidv = cstart + g * 16 + iota16
                    m01 = (_step01(y0 - (lo - 1)) * _step01((hi - 1) - y0)
                           * _step01(pidv - logical))
                    idxv = (y0 - (lo - 1)) * TS + x0
                    incl = plsc.cumsum(m01)
                    datas.append((m01, incl, pidv, fxv, fyv, idxv))
                for m01, incl, pidv, fxv, fyv, idxv in datas:
                    pos_sel = tail - 1 + incl
                    pos = TRASH + m01 * (pos_sel - TRASH)
                    plsc.store_scatter(pid_q, [pos], pidv)
                    plsc.store_scatter(fx_q, [pos], fxv)
                    plsc.store_scatter(fy_q, [pos], fyv)
                    plsc.store_scatter(idx_q, [pos], idxv)
                    tail = tail + incl[15]
                return tail
            tail = lax.fori_loop(0, S // 64, group, tail)

            nblk = tail // E
            pend = lax.fori_loop(0, nblk, do_emit, pend)
            for k in range(E // 16):
                off = nblk * E + k * 16
                pv = pid_q[pl.ds(off, 16)]
                fxv = fx_q[pl.ds(off, 16)]
                fyv = fy_q[pl.ds(off, 16)]
                iv = idx_q[pl.ds(off, 16)]
                pid_q[pl.ds(k * 16, 16)] = pv
                fx_q[pl.ds(k * 16, 16)] = fxv
                fy_q[pl.ds(k * 16, 16)] = fyv
                idx_q[pl.ds(k * 16, 16)] = iv
            return (tail - nblk * E, pend)

        issue_scan(0, 0)
        tail, pend = lax.fori_loop(0, T_CHUNKS, scan_chunk,
                                   (jnp.int32(0), jnp.int32(0)))
        wait_scan(T_CHUNKS - 1, lax.rem(jnp.int32(T_CHUNKS), 2))

        for k in range(E // 16):
            pid_q[pl.ds(tail + k * 16, 16)] = zi16
            fx_q[pl.ds(tail + k * 16, 16)] = zf16
            fy_q[pl.ds(tail + k * 16, 16)] = zf16
            idx_q[pl.ds(tail + k * 16, 16)] = zi16
        pend = lax.fori_loop(0, (tail + E - 1) // E, do_emit, pend)

        @pl.when(pend == 1)
        def _final_drain():
            for q in range(4):
                pltpu.make_async_copy(out_data.at[pl.ds(q * E, E)],
                                      acc.at[out_idx.at[q]], sem_s).wait()

        plsc.subcore_barrier()

    def run_slab(acc, rows, lo, hi):
        run_pass(acc, lo, hi)
        wrows = rows * (TS // 16)
        src = TS + sid * wrows
        dst = lo * TS + sid * wrows
        pltpu.sync_copy(acc.at[pl.ds(src, wrows)],
                        out_hbm.at[pl.ds(dst, wrows)])
        plsc.subcore_barrier()

    def body(acc):
        for p in range(5):
            slab = 2 * p + cid
            lo = slab * R
            hi = jnp.minimum(lo + R, TS)

            @pl.when(slab < NSLAB - 1)
            def _full():
                run_slab(acc, R, lo, hi)

            @pl.when(slab == NSLAB - 1)
            def _last():
                run_slab(acc, TS - (NSLAB - 1) * R, lo, hi)

    return body


def _splat_kernel(values_hbm, u_hbm, v_hbm, out_hbm,
                  u_buf, v_buf, pid_q, fx_q, fy_q, idx_q,
                  rows_v, out_data, out_idx, acc, sem_scan, sem_g, sem_s):
    body = _splat_body(values_hbm, u_hbm, v_hbm, out_hbm,
                       u_buf, v_buf, pid_q, fx_q, fy_q, idx_q,
                       rows_v, out_data, out_idx, sem_scan, sem_g, sem_s)
    body(acc)


def kernel(values_tensor, uv_tensor, texture_size):
    del texture_size
    u = uv_tensor[:, 0]
    v = uv_tensor[:, 1]

    mesh = plsc.VectorSubcoreMesh(core_axis_name="c", subcore_axis_name="s")
    fn = pl.kernel(
        _splat_kernel,
        out_type=jax.ShapeDtypeStruct((TS * TS, C), jnp.float32),
        mesh=mesh,
        scratch_types=[
            pltpu.VMEM((2 * S,), jnp.float32),
            pltpu.VMEM((2 * S,), jnp.float32),
            pltpu.VMEM((Q,), jnp.int32),
            pltpu.VMEM((Q,), jnp.float32),
            pltpu.VMEM((Q,), jnp.float32),
            pltpu.VMEM((Q,), jnp.int32),
            pltpu.VMEM((E, C), jnp.float32),
            pltpu.VMEM((4 * E, C), jnp.float32),
            pltpu.VMEM((4, E), jnp.int32),
            pltpu.VMEM_SHARED((ACC_ROWS, C), jnp.float32),
            pltpu.SemaphoreType.DMA,
            pltpu.SemaphoreType.DMA,
            pltpu.SemaphoreType.DMA,
        ],
        compiler_params=pltpu.CompilerParams(use_tc_tiling_on_sc=False,
                                             needs_layout_passes=False),
    )
    tex = fn(values_tensor, u, v)
    return tex.reshape(TS, TS, C)


# speedup vs baseline: 1.9025x; 1.1049x over previous
import jax
import jax.numpy as jnp
from jax import lax
from jax.experimental import pallas as pl
from jax.experimental.pallas import tpu as pltpu
from jax.experimental.pallas import tpu_sc as plsc

N = 1_000_000
C = 16
TS = 1024
R = 103
NSLAB = 10
ACC_ROWS = (R + 2) * TS
S = 1024
T_CHUNKS = 62
E = 128
Q = 1408
TRASH = Q - 16
ZROWS = 480
ACC_PER_TILE = ACC_ROWS // 16
PER_TILE = 62496


def _step01(x):
    return jnp.minimum(jnp.maximum(x + 1, 0), 1)


def _splat_body(values_hbm, u_hbm, v_hbm, out_hbm,
                u_buf, v_buf, pid_q, fx_q, fy_q, idx_q,
                rows_v, out_data, out_idx, sem_scan, sem_g, sem_s):
    cid = lax.axis_index("c")
    sid = lax.axis_index("s")

    start = sid * PER_TILE
    count = jnp.where(sid == 15, N - 15 * PER_TILE, PER_TILE)
    end = start + count
    iota16 = lax.iota(jnp.int32, 16)
    zf16 = jnp.zeros((16,), jnp.float32)
    zi16 = jnp.zeros((16,), jnp.int32)

    def chunk_start(i):
        return jnp.minimum(start + i * S, end - S)

    def issue_scan(i, par):
        cs = chunk_start(i)
        pltpu.async_copy(u_hbm.at[pl.ds(cs, S)],
                         u_buf.at[pl.ds(par * S, S)], sem_scan)
        pltpu.async_copy(v_hbm.at[pl.ds(cs, S)],
                         v_buf.at[pl.ds(par * S, S)], sem_scan)

    def wait_scan(i, par):
        cs = chunk_start(i)
        pltpu.make_async_copy(u_hbm.at[pl.ds(cs, S)],
                              u_buf.at[pl.ds(par * S, S)], sem_scan).wait()
        pltpu.make_async_copy(v_hbm.at[pl.ds(cs, S)],
                              v_buf.at[pl.ds(par * S, S)], sem_scan).wait()

    def run_pass(acc, lo, hi):
        def _zb(i, _):
            out_data[i, :] = zf16
            return 0
        lax.fori_loop(0, ZROWS, _zb, 0)
        zdescs = []
        for k in range(ACC_PER_TILE // ZROWS):
            zdescs.append(pltpu.async_copy(
                out_data.at[pl.ds(0, ZROWS)],
                acc.at[pl.ds(sid * ACC_PER_TILE + k * ZROWS, ZROWS)],
                sem_scan))
        for d in zdescs:
            d.wait()
        plsc.subcore_barrier()

        def issue_gather(e, par):
            pltpu.async_copy(values_hbm.at[pid_q.at[pl.ds(e * E, E)]],
                             rows_v.at[pl.ds(par * E, E)], sem_g)

        def wait_gather(e, par):
            pltpu.make_async_copy(values_hbm.at[pid_q.at[pl.ds(e * E, E)]],
                                  rows_v.at[pl.ds(par * E, E)], sem_g).wait()

        def drain(nblk, gpre, pend):
            @pl.when((nblk > 0) & (gpre == 0))
            def _prime():
                issue_gather(0, 0)

            def do_emit(e, pend):
                head = e * E
                par = lax.rem(e, 2)

                @pl.when(e + 1 < nblk)
                def _prefetch():
                    issue_gather(e + 1, lax.rem(e + 1, 2))

                @pl.when(pend == 1)
                def _drain_sc():
                    for q in range(4):
                        pltpu.make_async_copy(out_data.at[pl.ds(q * E, E)],
                                              acc.at[out_idx.at[q]],
                                              sem_s).wait()
                wait_gather(e, par)
                rbase = par * E

                def per_group(g, _):
                    base = head + g * 16
                    idxv = idx_q[pl.ds(base, 16)]
                    out_idx[0, pl.ds(g * 16, 16)] = idxv
                    out_idx[1, pl.ds(g * 16, 16)] = idxv + 1
                    out_idx[2, pl.ds(g * 16, 16)] = idxv + TS
                    out_idx[3, pl.ds(g * 16, 16)] = idxv + (TS + 1)
                    fxg = fx_q[pl.ds(base, 16)]
                    fyg = fy_q[pl.ds(base, 16)]
                    for j in range(16):
                        r = g * 16 + j
                        fx_s = fxg[j]
                        fy_s = fyg[j]
                        vrow = rows_v[rbase + r, :]
                        b = vrow * fy_s
                        a = vrow - b
                        t10 = a * fx_s
                        t00 = a - t10
                        t11 = b * fx_s
                        t01 = b - t11
                        out_data[r, :] = t00
                        out_data[E + r, :] = t10
                        out_data[2 * E + r, :] = t01
                        out_data[3 * E + r, :] = t11
                    return 0
                lax.fori_loop(0, E // 16, per_group, 0)

                for q in range(4):
                    pltpu.async_copy(out_data.at[pl.ds(q * E, E)],
                                     acc.at[out_idx.at[q]], sem_s, add=True)
                return jnp.int32(1)
            return lax.fori_loop(0, nblk, do_emit, pend)

        def scan_chunk(i, carry):
            tail, pend, gpre = carry
            par = lax.rem(i, 2)
            wait_scan(i, par)
            issue_scan(jnp.minimum(i + 1, T_CHUNKS - 1), lax.rem(i + 1, 2))
            cstart = chunk_start(i)
            logical = start + i * S
            brow = par * S

            def group(g4, tail):
                datas = []
                for uu in range(4):
                    g = g4 * 4 + uu
                    u16 = u_buf[pl.ds(brow + g * 16, 16)]
                    v16 = v_buf[pl.ds(brow + g * 16, 16)]
                    ux = u16 * jnp.float32(TS - 1)
                    vy = v16 * jnp.float32(TS - 1)
                    x0 = jnp.minimum(ux.astype(jnp.int32), TS - 2)
                    y0 = jnp.minimum(vy.astype(jnp.int32), TS - 2)
                    fxv = ux - x0.astype(jnp.float32)
                    fyv = vy - y0.astype(jnp.float32)
                    pidv = cstart + g * 16 + iota16
                    m01 = (_step01(y0 - (lo - 1)) * _step01((hi - 1) - y0)
                           * _step01(pidv - logical))
                    idxv = (y0 - (lo - 1)) * TS + x0
                    incl = plsc.cumsum(m01)
                    datas.append((m01, incl, pidv, fxv, fyv, idxv))
                for m01, incl, pidv, fxv, fyv, idxv in datas:
                    pos_sel = tail - 1 + incl
                    pos = TRASH + m01 * (pos_sel - TRASH)
                    plsc.store_scatter(pid_q, [pos], pidv)
                    plsc.store_scatter(fx_q, [pos], fxv)
                    plsc.store_scatter(fy_q, [pos], fyv)
                    plsc.store_scatter(idx_q, [pos], idxv)
                    tail = tail + incl[15]
                return tail
            tail = lax.fori_loop(0, S // 64, group, tail)

            nblk = jnp.maximum(tail // E - 1, 0)
            pend = drain(nblk, gpre, pend)
            for k in range(2 * E // 16):
                off = nblk * E + k * 16
                pv = pid_q[pl.ds(off, 16)]
                fxv = fx_q[pl.ds(off, 16)]
                fyv = fy_q[pl.ds(off, 16)]
                iv = idx_q[pl.ds(off, 16)]
                pid_q[pl.ds(k * 16, 16)] = pv
                fx_q[pl.ds(k * 16, 16)] = fxv
                fy_q[pl.ds(k * 16, 16)] = fyv
                idx_q[pl.ds(k * 16, 16)] = iv
            tail = tail - nblk * E
            keep = jnp.where(nblk == 0, gpre, 0)

            @pl.when((tail >= E) & (keep == 0))
            def _preissue():
                issue_gather(0, 0)
            gpre = jnp.where(tail >= E, 1, keep)
            return (tail, pend, gpre)

        issue_scan(0, 0)
        tail, pend, gpre = lax.fori_loop(
            0, T_CHUNKS, scan_chunk,
            (jnp.int32(0), jnp.int32(0), jnp.int32(0)))
        wait_scan(T_CHUNKS - 1, lax.rem(jnp.int32(T_CHUNKS), 2))

        for k in range(E // 16):
            pid_q[pl.ds(tail + k * 16, 16)] = zi16
            fx_q[pl.ds(tail + k * 16, 16)] = zf16
            fy_q[pl.ds(tail + k * 16, 16)] = zf16
            idx_q[pl.ds(tail + k * 16, 16)] = zi16
        pend = drain((tail + E - 1) // E, gpre, pend)

        @pl.when(pend == 1)
        def _final_drain():
            for q in range(4):
                pltpu.make_async_copy(out_data.at[pl.ds(q * E, E)],
                                      acc.at[out_idx.at[q]], sem_s).wait()

        plsc.subcore_barrier()

    def run_slab(acc, rows, lo, hi):
        run_pass(acc, lo, hi)
        wrows = rows * (TS // 16)
        src = TS + sid * wrows
        dst = lo * TS + sid * wrows
        pltpu.sync_copy(acc.at[pl.ds(src, wrows)],
                        out_hbm.at[pl.ds(dst, wrows)])
        plsc.subcore_barrier()

    def body(acc):
        for p in range(5):
            slab = 2 * p + cid
            lo = slab * R
            hi = jnp.minimum(lo + R, TS)

            @pl.when(slab < NSLAB - 1)
            def _full():
                run_slab(acc, R, lo, hi)

            @pl.when(slab == NSLAB - 1)
            def _last():
                run_slab(acc, TS - (NSLAB - 1) * R, lo, hi)

    return body


def _splat_kernel(values_hbm, u_hbm, v_hbm, out_hbm,
                  u_buf, v_buf, pid_q, fx_q, fy_q, idx_q,
                  rows_v, out_data, out_idx, acc, sem_scan, sem_g, sem_s):
    body = _splat_body(values_hbm, u_hbm, v_hbm, out_hbm,
                       u_buf, v_buf, pid_q, fx_q, fy_q, idx_q,
                       rows_v, out_data, out_idx, sem_scan, sem_g, sem_s)
    body(acc)


def kernel(values_tensor, uv_tensor, texture_size):
    del texture_size
    u = uv_tensor[:, 0]
    v = uv_tensor[:, 1]

    mesh = plsc.VectorSubcoreMesh(core_axis_name="c", subcore_axis_name="s")
    fn = pl.kernel(
        _splat_kernel,
        out_type=jax.ShapeDtypeStruct((TS * TS, C), jnp.float32),
        mesh=mesh,
        scratch_types=[
            pltpu.VMEM((2 * S,), jnp.float32),
            pltpu.VMEM((2 * S,), jnp.float32),
            pltpu.VMEM((Q,), jnp.int32),
            pltpu.VMEM((Q,), jnp.float32),
            pltpu.VMEM((Q,), jnp.float32),
            pltpu.VMEM((Q,), jnp.int32),
            pltpu.VMEM((2 * E, C), jnp.float32),
            pltpu.VMEM((4 * E, C), jnp.float32),
            pltpu.VMEM((4, E), jnp.int32),
            pltpu.VMEM_SHARED((ACC_ROWS, C), jnp.float32),
            pltpu.SemaphoreType.DMA,
            pltpu.SemaphoreType.DMA,
            pltpu.SemaphoreType.DMA,
        ],
        compiler_params=pltpu.CompilerParams(use_tc_tiling_on_sc=False,
                                             needs_layout_passes=False),
    )
    tex = fn(values_tensor, u, v)
    return tex.reshape(TS, TS, C)
